# baseline (speedup 1.0000x reference)
import jax
import jax.numpy as jnp
from jax import lax
from jax.experimental import pallas as pl
from jax.experimental.pallas import tpu as pltpu
from jax.experimental.pallas import tpu_sc as plsc

N_ENTITY = 100000
N_RELATION = 1000
D = 64
B = 16384
NC = 2
NS = 16
NW = NC * NS
PER_W = B // NW
C = 128
NCHUNK = PER_W // C
NV = D // 16
R_PER = 64


def _rsqrt(x, iters):
    x = jnp.maximum(x, 1e-12)
    i = lax.bitcast_convert_type(x, jnp.int32)
    i = jnp.int32(0x5F3759DF) - (i >> 1)
    y = lax.bitcast_convert_type(i, jnp.float32)
    for _ in range(iters):
        y = y * (1.5 - 0.5 * x * y * y)
    return y


def _sc_body(ent, rel, nrm, heads_p, tails_p, rels_p, heads_n, tails_n,
             rels_n, out_p, out_n, rhat, nhat, idx_h, idx_t, idx_r,
             hbuf, tbuf, rbuf, nbuf, obuf, obuf_t, sem):
    cid = lax.axis_index("c")
    sid = lax.axis_index("s")
    wid = sid * NC + cid
    rel_off = cid * N_RELATION

    def norm_rows(src, dst, buf, n_rows, r0):
        pltpu.sync_copy(src.at[pl.ds(r0, n_rows)], buf.at[pl.ds(0, n_rows)])

        def row_body(r, _):
            rv = [buf[r, pl.ds(16 * k, 16)] for k in range(NV)]
            s = rv[0] * rv[0]
            for k in range(1, NV):
                s = s + rv[k] * rv[k]
            e = _rsqrt(jnp.sum(s), 3)
            for k in range(NV):
                buf[r, pl.ds(16 * k, 16)] = e * rv[k]
            return _

        lax.fori_loop(0, n_rows, row_body, None)
        pltpu.sync_copy(buf.at[pl.ds(0, n_rows)],
                        dst.at[pl.ds(rel_off + r0, n_rows)])

    @pl.when(sid < NS - 1)
    def _full():
        r0 = pl.multiple_of(sid * R_PER, R_PER)
        norm_rows(rel, rhat, rbuf, R_PER, r0)
        norm_rows(nrm, nhat, nbuf, R_PER, r0)

    @pl.when(sid == NS - 1)
    def _tail():
        r0 = (NS - 1) * R_PER
        norm_rows(rel, rhat, rbuf, N_RELATION - r0, r0)
        norm_rows(nrm, nhat, nbuf, N_RELATION - r0, r0)

    plsc.subcore_barrier()

    rows_k = [16 * k + lax.iota(jnp.int32, 16) for k in range(NV)]

    def compute_triple(i, _):
        cols = jnp.full((16,), i, jnp.int32)
        hv = [hbuf[i, pl.ds(16 * k, 16)] for k in range(NV)]
        tv = [tbuf[i, pl.ds(16 * k, 16)] for k in range(NV)]
        rv = [rbuf[i, pl.ds(16 * k, 16)] for k in range(NV)]
        nv = [nbuf[i, pl.ds(16 * k, 16)] for k in range(NV)]
        sh = hv[0] * hv[0]
        st = tv[0] * tv[0]
        dh = hv[0] * nv[0]
        dt = tv[0] * nv[0]
        for k in range(1, NV):
            sh = sh + hv[k] * hv[k]
            st = st + tv[k] * tv[k]
            dh = dh + hv[k] * nv[k]
            dt = dt + tv[k] * nv[k]
        a = _rsqrt(jnp.sum(sh), 2)
        b = _rsqrt(jnp.sum(st), 2)
        g = a * jnp.sum(dh) - b * jnp.sum(dt)
        for k in range(NV):
            obuf[i, pl.ds(16 * k, 16)] = (
                a * hv[k] - b * tv[k] + rv[k] - g * nv[k])
        return _

    def process(heads, tails, rels, out):
        for j in range(NCHUNK):
            base = wid * PER_W + j * C
            pltpu.sync_copy(heads.at[pl.ds(base, C)], idx_h)
            pltpu.sync_copy(tails.at[pl.ds(base, C)], idx_t)
            pltpu.sync_copy(rels.at[pl.ds(base, C)], idx_r)
            for k in range(C // 16):
                s = pl.ds(k * 16, 16)
                idx_r[s] = idx_r[s] + rel_off
            d1 = pltpu.async_copy(ent.at[idx_h], hbuf, sem)
            d2 = pltpu.async_copy(ent.at[idx_t], tbuf, sem)
            d3 = pltpu.async_copy(rhat.at[idx_r], rbuf, sem)
            d4 = pltpu.async_copy(nhat.at[idx_r], nbuf, sem)
            d1.wait()
            d2.wait()
            d3.wait()
            d4.wait()
            lax.fori_loop(0, C, compute_triple, None)

            def transpose_col(i, _):
                cols = jnp.full((16,), i, jnp.int32)
                for k in range(NV):
                    plsc.store_scatter(obuf_t, [rows_k[k], cols],
                                       obuf[i, pl.ds(16 * k, 16)])
                return _

            lax.fori_loop(0, C, transpose_col, None)
            pltpu.sync_copy(obuf_t, out.at[:, pl.ds(base, C)])

    process(heads_p, tails_p, rels_p, out_p)
    process(heads_n, tails_n, rels_n, out_n)


@jax.jit
def kernel(entity_embedding, relation_embedding, normal_embedding,
           heads_pos, tails_pos, rels_pos,
           heads_neg, tails_neg, rels_neg):
    mesh = plsc.VectorSubcoreMesh(core_axis_name="c", subcore_axis_name="s")
    run = pl.kernel(
        _sc_body,
        mesh=mesh,
        compiler_params=pltpu.CompilerParams(
            needs_layout_passes=False, use_tc_tiling_on_sc=False),
        out_type=(
            jax.ShapeDtypeStruct((D, B), jnp.float32),
            jax.ShapeDtypeStruct((D, B), jnp.float32),
            jax.ShapeDtypeStruct((NC * N_RELATION, D), jnp.float32),
            jax.ShapeDtypeStruct((NC * N_RELATION, D), jnp.float32),
        ),
        scratch_types=[
            pltpu.VMEM((C,), jnp.int32),
            pltpu.VMEM((C,), jnp.int32),
            pltpu.VMEM((C,), jnp.int32),
            pltpu.VMEM((C, D), jnp.float32),
            pltpu.VMEM((C, D), jnp.float32),
            pltpu.VMEM((C, D), jnp.float32),
            pltpu.VMEM((C, D), jnp.float32),
            pltpu.VMEM((C, D), jnp.float32),
            pltpu.VMEM((D, C), jnp.float32),
            pltpu.SemaphoreType.DMA,
        ],
    )
    out_tp, out_tn = run(entity_embedding, relation_embedding,
                         normal_embedding,
                         heads_pos, tails_pos, rels_pos,
                         heads_neg, tails_neg, rels_neg)[:2]
    return out_tp.T, out_tn.T


# speedup vs baseline: 1.2406x; 1.2406x over previous
import jax
import jax.numpy as jnp
from jax import lax
from jax.experimental import pallas as pl
from jax.experimental.pallas import tpu as pltpu
from jax.experimental.pallas import tpu_sc as plsc

N_ENTITY = 100000
N_RELATION = 1000
D = 64
B = 16384
NC = 2
NS = 16
NW = NC * NS
PER_W = B // NW
C = 128
NCHUNK = PER_W // C
NV = D // 16
R_PER = 64


def _rsqrt(x, iters):
    x = jnp.maximum(x, 1e-12)
    i = lax.bitcast_convert_type(x, jnp.int32)
    i = jnp.int32(0x5F3759DF) - (i >> 1)
    y = lax.bitcast_convert_type(i, jnp.float32)
    for _ in range(iters):
        y = y * (1.5 - 0.5 * x * y * y)
    return y


def _sc_body(ent, rel, nrm, heads_p, tails_p, rels_p, heads_n, tails_n,
             rels_n, out_p, out_n, rhat, nhat,
             ih0, it0, ir0, ih1, it1, ir1,
             hb0, tb0, rb0, nb0, hb1, tb1, rb1, nb1, obuf, sem0, sem1):
    cid = lax.axis_index("c")
    sid = lax.axis_index("s")
    wid = sid * NC + cid
    rel_off = cid * N_RELATION
    idxs = [(ih0, it0, ir0), (ih1, it1, ir1)]
    bufs = [(hb0, tb0, rb0, nb0), (hb1, tb1, rb1, nb1)]
    sems = [sem0, sem1]

    def norm_rows(src, dst, buf, n_rows, r0):
        pltpu.sync_copy(src.at[pl.ds(r0, n_rows)], buf.at[pl.ds(0, n_rows)])

        def row_body(r, _):
            rv = [buf[r, pl.ds(16 * k, 16)] for k in range(NV)]
            s = rv[0] * rv[0]
            for k in range(1, NV):
                s = s + rv[k] * rv[k]
            e = _rsqrt(jnp.sum(s), 3)
            for k in range(NV):
                buf[r, pl.ds(16 * k, 16)] = e * rv[k]
            return _

        lax.fori_loop(0, n_rows, row_body, None)
        pltpu.sync_copy(buf.at[pl.ds(0, n_rows)],
                        dst.at[pl.ds(rel_off + r0, n_rows)])

    @pl.when(sid < NS - 1)
    def _full():
        r0 = pl.multiple_of(sid * R_PER, R_PER)
        norm_rows(rel, rhat, rb0, R_PER, r0)
        norm_rows(nrm, nhat, nb0, R_PER, r0)

    @pl.when(sid == NS - 1)
    def _tail():
        r0 = (NS - 1) * R_PER
        norm_rows(rel, rhat, rb0, N_RELATION - r0, r0)
        norm_rows(nrm, nhat, nb0, N_RELATION - r0, r0)

    plsc.subcore_barrier()

    def make_compute(hbuf, tbuf, rbuf, nbuf):
        def compute_triple(i, _):
            hv = [hbuf[i, pl.ds(16 * k, 16)] for k in range(NV)]
            tv = [tbuf[i, pl.ds(16 * k, 16)] for k in range(NV)]
            rv = [rbuf[i, pl.ds(16 * k, 16)] for k in range(NV)]
            nv = [nbuf[i, pl.ds(16 * k, 16)] for k in range(NV)]
            sh = hv[0] * hv[0]
            st = tv[0] * tv[0]
            dh = hv[0] * nv[0]
            dt = tv[0] * nv[0]
            for k in range(1, NV):
                sh = sh + hv[k] * hv[k]
                st = st + tv[k] * tv[k]
                dh = dh + hv[k] * nv[k]
                dt = dt + tv[k] * nv[k]
            a = _rsqrt(jnp.sum(sh), 2)
            b = _rsqrt(jnp.sum(st), 2)
            g = a * jnp.sum(dh) - b * jnp.sum(dt)
            for k in range(NV):
                obuf[i, pl.ds(16 * k, 16)] = (
                    a * hv[k] - b * tv[k] + rv[k] - g * nv[k])
            return _
        return compute_triple

    chunks = [(heads_p, tails_p, rels_p, out_p, j) for j in range(NCHUNK)]
    chunks += [(heads_n, tails_n, rels_n, out_n, j) for j in range(NCHUNK)]

    def start_gathers(sel, heads, tails, rels, j):
        base = wid * PER_W + j * C
        idx_h, idx_t, idx_r = idxs[sel]
        hb, tb, rb, nb = bufs[sel]
        pltpu.sync_copy(heads.at[pl.ds(base, C)], idx_h)
        pltpu.sync_copy(tails.at[pl.ds(base, C)], idx_t)
        pltpu.sync_copy(rels.at[pl.ds(base, C)], idx_r)
        for k in range(C // 16):
            s = pl.ds(k * 16, 16)
            idx_r[s] = idx_r[s] + rel_off
        return [pltpu.async_copy(ent.at[idx_h], hb, sems[sel]),
                pltpu.async_copy(ent.at[idx_t], tb, sems[sel]),
                pltpu.async_copy(rhat.at[idx_r], rb, sems[sel]),
                pltpu.async_copy(nhat.at[idx_r], nb, sems[sel])]

    descs = start_gathers(0, *chunks[0][:3], chunks[0][4])
    for ci, (heads, tails, rels, out, j) in enumerate(chunks):
        sel = ci % 2
        for d in descs:
            d.wait()
        if ci + 1 < len(chunks):
            nxt = chunks[ci + 1]
            descs = start_gathers(1 - sel, *nxt[:3], nxt[4])
        hb, tb, rb, nb = bufs[sel]
        lax.fori_loop(0, C, make_compute(hb, tb, rb, nb), None)
        base = wid * PER_W + j * C
        pltpu.sync_copy(obuf, out.at[pl.ds(base, C)])


@jax.jit
def kernel(entity_embedding, relation_embedding, normal_embedding,
           heads_pos, tails_pos, rels_pos,
           heads_neg, tails_neg, rels_neg):
    mesh = plsc.VectorSubcoreMesh(core_axis_name="c", subcore_axis_name="s")
    run = pl.kernel(
        _sc_body,
        mesh=mesh,
        compiler_params=pltpu.CompilerParams(
            needs_layout_passes=False, use_tc_tiling_on_sc=False),
        out_type=(
            jax.ShapeDtypeStruct((B, D), jnp.float32),
            jax.ShapeDtypeStruct((B, D), jnp.float32),
            jax.ShapeDtypeStruct((NC * N_RELATION, D), jnp.float32),
            jax.ShapeDtypeStruct((NC * N_RELATION, D), jnp.float32),
        ),
        scratch_types=(
            [pltpu.VMEM((C,), jnp.int32)] * 6
            + [pltpu.VMEM((C, D), jnp.float32)] * 9
            + [pltpu.SemaphoreType.DMA] * 2
        ),
    )
    out_p, out_n = run(entity_embedding, relation_embedding,
                       normal_embedding,
                       heads_pos, tails_pos, rels_pos,
                       heads_neg, tails_neg, rels_neg)[:2]
    return out_p, out_n


# speedup vs baseline: 1.2496x; 1.0072x over previous
import jax
import jax.numpy as jnp
from jax import lax
from jax.experimental import pallas as pl
from jax.experimental.pallas import tpu as pltpu
from jax.experimental.pallas import tpu_sc as plsc

N_ENTITY = 100000
N_RELATION = 1000
D = 64
B = 16384
NC = 2
NS = 16
NW = NC * NS
PER_W = B // NW
C = 128
NCHUNK = PER_W // C
NV = D // 16
R_PER = 64


def _rsqrt(x, iters):
    x = jnp.maximum(x, 1e-12)
    i = lax.bitcast_convert_type(x, jnp.int32)
    i = jnp.int32(0x5F3759DF) - (i >> 1)
    y = lax.bitcast_convert_type(i, jnp.float32)
    for _ in range(iters):
        y = y * (1.5 - 0.5 * x * y * y)
    return y


def _sc_body(ent, rel, nrm, heads_p, tails_p, rels_p, heads_n, tails_n,
             rels_n, out_p, out_n, rhat, nhat,
             ih0, it0, ir0, ih1, it1, ir1,
             hb0, tb0, rb0, nb0, hb1, tb1, rb1, nb1, ob0, ob1,
             sem0, sem1, osem):
    cid = lax.axis_index("c")
    sid = lax.axis_index("s")
    wid = sid * NC + cid
    rel_off = cid * N_RELATION
    idxs = [(ih0, it0, ir0), (ih1, it1, ir1)]
    bufs = [(hb0, tb0, rb0, nb0), (hb1, tb1, rb1, nb1)]
    sems = [sem0, sem1]
    obufs = [ob0, ob1]

    def norm_rows(src, dst, buf, n_rows, r0):
        pltpu.sync_copy(src.at[pl.ds(r0, n_rows)], buf.at[pl.ds(0, n_rows)])

        def row_body(r, _):
            rv = [buf[r, pl.ds(16 * k, 16)] for k in range(NV)]
            s = rv[0] * rv[0]
            for k in range(1, NV):
                s = s + rv[k] * rv[k]
            e = _rsqrt(jnp.sum(s), 3)
            for k in range(NV):
                buf[r, pl.ds(16 * k, 16)] = e * rv[k]
            return _

        lax.fori_loop(0, n_rows, row_body, None)
        pltpu.sync_copy(buf.at[pl.ds(0, n_rows)],
                        dst.at[pl.ds(rel_off + r0, n_rows)])

    @pl.when(sid < NS - 1)
    def _full():
        r0 = pl.multiple_of(sid * R_PER, R_PER)
        norm_rows(rel, rhat, rb0, R_PER, r0)
        norm_rows(nrm, nhat, nb0, R_PER, r0)

    @pl.when(sid == NS - 1)
    def _tail():
        r0 = (NS - 1) * R_PER
        norm_rows(rel, rhat, rb0, N_RELATION - r0, r0)
        norm_rows(nrm, nhat, nb0, N_RELATION - r0, r0)

    plsc.subcore_barrier()

    odescs = [None, None]

    def make_compute(hbuf, tbuf, rbuf, nbuf, obuf):
        def compute_triple(i, _):
            hv = [hbuf[i, pl.ds(16 * k, 16)] for k in range(NV)]
            tv = [tbuf[i, pl.ds(16 * k, 16)] for k in range(NV)]
            rv = [rbuf[i, pl.ds(16 * k, 16)] for k in range(NV)]
            nv = [nbuf[i, pl.ds(16 * k, 16)] for k in range(NV)]
            sh = hv[0] * hv[0]
            st = tv[0] * tv[0]
            dh = hv[0] * nv[0]
            dt = tv[0] * nv[0]
            for k in range(1, NV):
                sh = sh + hv[k] * hv[k]
                st = st + tv[k] * tv[k]
                dh = dh + hv[k] * nv[k]
                dt = dt + tv[k] * nv[k]
            a = _rsqrt(jnp.sum(sh), 2)
            b = _rsqrt(jnp.sum(st), 2)
            g = a * jnp.sum(dh) - b * jnp.sum(dt)
            for k in range(NV):
                obuf[i, pl.ds(16 * k, 16)] = (
                    a * hv[k] - b * tv[k] + rv[k] - g * nv[k])
            return _
        return compute_triple

    chunks = [(heads_p, tails_p, rels_p, out_p, j) for j in range(NCHUNK)]
    chunks += [(heads_n, tails_n, rels_n, out_n, j) for j in range(NCHUNK)]

    def start_gathers(sel, heads, tails, rels, j):
        base = wid * PER_W + j * C
        idx_h, idx_t, idx_r = idxs[sel]
        hb, tb, rb, nb = bufs[sel]
        pltpu.sync_copy(heads.at[pl.ds(base, C)], idx_h)
        pltpu.sync_copy(tails.at[pl.ds(base, C)], idx_t)
        pltpu.sync_copy(rels.at[pl.ds(base, C)], idx_r)
        for k in range(C // 16):
            s = pl.ds(k * 16, 16)
            idx_r[s] = idx_r[s] + rel_off
        return [pltpu.async_copy(ent.at[idx_h], hb, sems[sel]),
                pltpu.async_copy(ent.at[idx_t], tb, sems[sel]),
                pltpu.async_copy(rhat.at[idx_r], rb, sems[sel]),
                pltpu.async_copy(nhat.at[idx_r], nb, sems[sel])]

    descs = start_gathers(0, *chunks[0][:3], chunks[0][4])
    for ci, (heads, tails, rels, out, j) in enumerate(chunks):
        sel = ci % 2
        for d in descs:
            d.wait()
        if ci + 1 < len(chunks):
            nxt = chunks[ci + 1]
            descs = start_gathers(1 - sel, *nxt[:3], nxt[4])
        hb, tb, rb, nb = bufs[sel]
        if odescs[sel] is not None:
            odescs[sel].wait()
        lax.fori_loop(0, C, make_compute(hb, tb, rb, nb, obufs[sel]), None)
        base = wid * PER_W + j * C
        odescs[sel] = pltpu.async_copy(obufs[sel], out.at[pl.ds(base, C)],
                                       osem)
    for dsc in odescs:
        if dsc is not None:
            dsc.wait()


@jax.jit
def kernel(entity_embedding, relation_embedding, normal_embedding,
           heads_pos, tails_pos, rels_pos,
           heads_neg, tails_neg, rels_neg):
    mesh = plsc.VectorSubcoreMesh(core_axis_name="c", subcore_axis_name="s")
    run = pl.kernel(
        _sc_body,
        mesh=mesh,
        compiler_params=pltpu.CompilerParams(
            needs_layout_passes=False, use_tc_tiling_on_sc=False),
        out_type=(
            jax.ShapeDtypeStruct((B, D), jnp.float32),
            jax.ShapeDtypeStruct((B, D), jnp.float32),
            jax.ShapeDtypeStruct((NC * N_RELATION, D), jnp.float32),
            jax.ShapeDtypeStruct((NC * N_RELATION, D), jnp.float32),
        ),
        scratch_types=(
            [pltpu.VMEM((C,), jnp.int32)] * 6
            + [pltpu.VMEM((C, D), jnp.float32)] * 10
            + [pltpu.SemaphoreType.DMA] * 3
        ),
    )
    out_p, out_n = run(entity_embedding, relation_embedding,
                       normal_embedding,
                       heads_pos, tails_pos, rels_pos,
                       heads_neg, tails_neg, rels_neg)[:2]
    return out_p, out_n
